# Initial kernel scaffold; baseline (speedup 1.0000x reference)
import jax
import jax.numpy as jnp
from jax.experimental import pallas as pl


def kernel(A_edge_index, A_values, X, w):
    raise NotImplementedError("write your pallas kernel here")


# speedup vs baseline: 1.5299x; 1.5299x over previous
import functools

import jax
import jax.numpy as jnp
from jax import lax
from jax.experimental import pallas as pl
from jax.experimental.pallas import tpu as pltpu
from jax.experimental.pallas import tpu_sc as plsc

_NC = 2
_NS = 16
_L = 16

_B = 128


def _sc_supports(src, dst, vals, X, n, n_pad, d, r_total, e_pad):
    r_per_core = r_total // _NC
    e_per_tile = e_pad // _NS
    nbatch = e_per_tile // _B
    rows_per_tile = n_pad // _NS
    nsl = d // _L
    zrows = 16

    mesh = plsc.VectorSubcoreMesh(core_axis_name="c", subcore_axis_name="s")

    @functools.partial(
        pl.kernel,
        out_type=jax.ShapeDtypeStruct((r_total, n_pad, d), jnp.float32),
        mesh=mesh,
        scratch_types=[
            pltpu.VMEM((zrows, d), jnp.float32),
            pltpu.VMEM((_B, d), jnp.float32),
            pltpu.VMEM((_B,), jnp.int32),
            pltpu.VMEM((_B,), jnp.int32),
            pltpu.VMEM((_B * _L,), jnp.float32),
            pltpu.VMEM_SHARED((n_pad, d), jnp.float32),
            pltpu.SemaphoreType.DMA,
        ],
    )
    def body(src_hbm, dst_hbm, val_hbm, x_hbm, sup_hbm,
             zbuf, rows, sidx, didx, vbuf, acc, sem):
        cid = lax.axis_index("c")
        sid = lax.axis_index("s")
        row0 = sid * rows_per_tile

        zero_v = jnp.zeros((_L,), jnp.float32)

        def zfill(i, _):
            e = i // nsl
            j = i % nsl
            zbuf[e, pl.ds(j * _L, _L)] = zero_v
            return 0

        lax.fori_loop(0, zrows * nsl, zfill, 0)

        for rr in range(r_per_core):
            r = cid * r_per_core + rr

            def zero_acc(z, _):
                pltpu.sync_copy(zbuf, acc.at[pl.ds(row0 + z * zrows, zrows)])
                return 0

            lax.fori_loop(0, rows_per_tile // zrows, zero_acc, 0)
            plsc.subcore_barrier()

            def batch_body(b, _):
                base = r * e_pad + sid * e_per_tile + b * _B
                pltpu.sync_copy(src_hbm.at[pl.ds(base, _B)], sidx)
                pltpu.sync_copy(dst_hbm.at[pl.ds(base, _B)], didx)
                pltpu.sync_copy(val_hbm.at[pl.ds(base * _L, _B * _L)], vbuf)
                pltpu.async_copy(x_hbm.at[sidx], rows, sem).wait()

                def scale_body(e, _):
                    vb = vbuf[pl.ds(e * _L, _L)]
                    for j in range(nsl):
                        sl = rows[e, pl.ds(j * _L, _L)]
                        rows[e, pl.ds(j * _L, _L)] = sl * vb
                    return 0

                lax.fori_loop(0, _B, scale_body, 0)

                pltpu.sync_copy(rows, acc.at[didx], add=True)
                return 0

            lax.fori_loop(0, nbatch, batch_body, 0)
            plsc.subcore_barrier()

            pltpu.sync_copy(acc.at[pl.ds(row0, rows_per_tile)],
                            sup_hbm.at[r, pl.ds(row0, rows_per_tile)])
            plsc.subcore_barrier()

    return body(src, dst, vals, X)


def _tc_matmul(sup, w, n, d, r_total, o):
    bn = 1000
    assert n % bn == 0

    def body(sup_ref, w_ref, out_ref):
        acc = jnp.zeros((bn, o), jnp.float32)
        for r in range(r_total):
            acc = acc + jnp.dot(sup_ref[r], w_ref[r],
                                preferred_element_type=jnp.float32)
        out_ref[...] = acc

    return pl.pallas_call(
        body,
        grid=(n // bn,),
        in_specs=[
            pl.BlockSpec((r_total, bn, d), lambda i: (0, i, 0)),
            pl.BlockSpec((r_total, d, o), lambda i: (0, 0, 0)),
        ],
        out_specs=pl.BlockSpec((bn, o), lambda i: (i, 0)),
        out_shape=jax.ShapeDtypeStruct((n, o), jnp.float32),
    )(sup, w)


def kernel(A_edge_index, A_values, X, w):
    n, d = X.shape
    r_total, _, e = A_edge_index.shape
    o = w.shape[2]

    chunk = _NS * _B
    e_pad = ((e + chunk - 1) // chunk) * chunk
    pad = e_pad - e
    dst = A_edge_index[:, 0, :]
    src = A_edge_index[:, 1, :]
    if pad:
        dst = jnp.pad(dst, ((0, 0), (0, pad)))
        src = jnp.pad(src, ((0, 0), (0, pad)))
        vals = jnp.pad(A_values, ((0, 0), (0, pad)))
    else:
        vals = A_values

    n_pad = ((n + _NS * 16 - 1) // (_NS * 16)) * (_NS * 16)

    vals_exp = jnp.repeat(vals.reshape(-1), _L)

    sup = _sc_supports(src.reshape(-1), dst.reshape(-1), vals_exp,
                       X, n, n_pad, d, r_total, e_pad)
    return _tc_matmul(sup, w, n, d, r_total, o)


# speedup vs baseline: 1.9154x; 1.2520x over previous
import functools

import jax
import jax.numpy as jnp
from jax import lax
from jax.experimental import pallas as pl
from jax.experimental.pallas import tpu as pltpu
from jax.experimental.pallas import tpu_sc as plsc

_NC = 2
_NS = 16
_L = 16

_B = 128


def _sc_supports(src, dst, vals_exp, X, n, n_pad, d, r_total, e_pad):
    r_per_core = r_total // _NC
    e_per_tile = e_pad // _NS
    nbatch = e_per_tile // _B
    assert nbatch % 2 == 0
    rows_per_tile = n_pad // _NS
    nsl = d // _L
    zrows = 16
    blv = _B * _L

    mesh = plsc.VectorSubcoreMesh(core_axis_name="c", subcore_axis_name="s")

    @functools.partial(
        pl.kernel,
        out_type=jax.ShapeDtypeStruct((r_total, n_pad, d), jnp.float32),
        mesh=mesh,
        scratch_types=[
            pltpu.VMEM((zrows, d), jnp.float32),
            pltpu.VMEM((nbatch, _B), jnp.int32),
            pltpu.VMEM((nbatch, _B), jnp.int32),
            pltpu.VMEM((_B, d), jnp.float32),
            pltpu.VMEM((_B, d), jnp.float32),
            pltpu.VMEM((blv,), jnp.float32),
            pltpu.VMEM((blv,), jnp.float32),
            pltpu.VMEM_SHARED((n_pad, d), jnp.float32),
            pltpu.SemaphoreType.DMA,
            pltpu.SemaphoreType.DMA,
            pltpu.SemaphoreType.DMA,
            pltpu.SemaphoreType.DMA,
        ],
    )
    def body(src_hbm, dst_hbm, val_hbm, x_hbm, sup_hbm,
             zbuf, sidx_all, didx_all, rows0, rows1, vbuf0, vbuf1, acc,
             gsem0, gsem1, vsem0, vsem1):
        cid = lax.axis_index("c")
        sid = lax.axis_index("s")
        row0 = sid * rows_per_tile
        rows = (rows0, rows1)
        vbufs = (vbuf0, vbuf1)
        gsems = (gsem0, gsem1)
        vsems = (vsem0, vsem1)

        zero_v = jnp.zeros((_L,), jnp.float32)

        def zfill(i, _):
            e = i // nsl
            j = i % nsl
            zbuf[e, pl.ds(j * _L, _L)] = zero_v
            return 0

        lax.fori_loop(0, zrows * nsl, zfill, 0)

        for rr in range(r_per_core):
            r = cid * r_per_core + rr
            vbase_rel = (r * _NS + sid) * nbatch * blv

            def zero_acc(z, _):
                pltpu.sync_copy(zbuf, acc.at[pl.ds(row0 + z * zrows, zrows)])
                return 0

            lax.fori_loop(0, rows_per_tile // zrows, zero_acc, 0)

            pltpu.sync_copy(src_hbm.at[r, sid], sidx_all)
            pltpu.sync_copy(dst_hbm.at[r, sid], didx_all)
            plsc.subcore_barrier()

            pltpu.async_copy(x_hbm.at[sidx_all.at[0]], rows0, gsem0)
            pltpu.async_copy(val_hbm.at[pl.ds(vbase_rel, blv)], vbuf0, vsem0)

            def pair_body(g, _):
                for p in range(2):
                    b = 2 * g + p
                    nb = b + 1

                    @pl.when(nb < nbatch)
                    def _():
                        pltpu.async_copy(x_hbm.at[sidx_all.at[nb]],
                                         rows[1 - p], gsems[1 - p])
                        pltpu.async_copy(
                            val_hbm.at[pl.ds(vbase_rel + nb * blv, blv)],
                            vbufs[1 - p], vsems[1 - p])

                    pltpu.make_async_copy(x_hbm.at[sidx_all.at[b]],
                                          rows[p], gsems[p]).wait()
                    pltpu.make_async_copy(
                        val_hbm.at[pl.ds(vbase_rel + b * blv, blv)],
                        vbufs[p], vsems[p]).wait()

                    def scale_body(e, _):
                        vb = vbufs[p][pl.ds(e * _L, _L)]
                        for j in range(nsl):
                            sl = rows[p][e, pl.ds(j * _L, _L)]
                            rows[p][e, pl.ds(j * _L, _L)] = sl * vb
                        return 0

                    lax.fori_loop(0, _B, scale_body, 0)

                    pltpu.sync_copy(rows[p], acc.at[didx_all.at[b]], add=True)
                return 0

            lax.fori_loop(0, nbatch // 2, pair_body, 0)
            plsc.subcore_barrier()

            pltpu.sync_copy(acc.at[pl.ds(row0, rows_per_tile)],
                            sup_hbm.at[r, pl.ds(row0, rows_per_tile)])
            plsc.subcore_barrier()

    return body(src, dst, vals_exp, X)


def _tc_matmul(sup, w, n, d, r_total, o):
    bn = 1000
    assert n % bn == 0

    def body(sup_ref, w_ref, out_ref):
        acc = jnp.zeros((bn, o), jnp.float32)
        for r in range(r_total):
            acc = acc + jnp.dot(sup_ref[r], w_ref[r],
                                preferred_element_type=jnp.float32)
        out_ref[...] = acc

    return pl.pallas_call(
        body,
        grid=(n // bn,),
        in_specs=[
            pl.BlockSpec((r_total, bn, d), lambda i: (0, i, 0)),
            pl.BlockSpec((r_total, d, o), lambda i: (0, 0, 0)),
        ],
        out_specs=pl.BlockSpec((bn, o), lambda i: (i, 0)),
        out_shape=jax.ShapeDtypeStruct((n, o), jnp.float32),
    )(sup, w)


def kernel(A_edge_index, A_values, X, w):
    n, d = X.shape
    r_total, _, e = A_edge_index.shape
    o = w.shape[2]

    chunk = _NS * _B * 2
    e_pad = ((e + chunk - 1) // chunk) * chunk
    pad = e_pad - e
    dst = A_edge_index[:, 0, :]
    src = A_edge_index[:, 1, :]
    if pad:
        dst = jnp.pad(dst, ((0, 0), (0, pad)))
        src = jnp.pad(src, ((0, 0), (0, pad)))
        vals = jnp.pad(A_values, ((0, 0), (0, pad)))
    else:
        vals = A_values

    n_pad = ((n + _NS * 16 - 1) // (_NS * 16)) * (_NS * 16)

    vals_exp = jnp.repeat(vals.reshape(-1), _L)

    nbatch = e_pad // (_NS * _B)
    src4 = src.reshape(r_total, _NS, nbatch, _B)
    dst4 = dst.reshape(r_total, _NS, nbatch, _B)

    sup = _sc_supports(src4, dst4, vals_exp, X, n, n_pad, d, r_total, e_pad)
    return _tc_matmul(sup, w, n, d, r_total, o)


# speedup vs baseline: 1.9690x; 1.0280x over previous
import functools

import jax
import jax.numpy as jnp
from jax import lax
from jax.experimental import pallas as pl
from jax.experimental.pallas import tpu as pltpu
from jax.experimental.pallas import tpu_sc as plsc

_NC = 2
_NS = 16
_L = 16

_B = 128


def _sc_supports(src, dst, vals_exp, X, n, n_pad, d, r_total, e_pad):
    r_per_core = r_total // _NC
    e_per_tile = e_pad // _NS
    nbatch = e_per_tile // _B
    assert nbatch % 2 == 0
    rows_per_tile = n_pad // _NS
    nsl = d // _L
    zrows = 16
    blv = _B * _L

    mesh = plsc.VectorSubcoreMesh(core_axis_name="c", subcore_axis_name="s")

    @functools.partial(
        pl.kernel,
        out_type=jax.ShapeDtypeStruct((r_total, n_pad, d), jnp.float32),
        mesh=mesh,
        scratch_types=[
            pltpu.VMEM((zrows, d), jnp.float32),
            pltpu.VMEM((nbatch, _B), jnp.int32),
            pltpu.VMEM((nbatch, _B), jnp.int32),
            pltpu.VMEM((_B, d), jnp.float32),
            pltpu.VMEM((_B, d), jnp.float32),
            pltpu.VMEM((blv,), jnp.float32),
            pltpu.VMEM((blv,), jnp.float32),
            pltpu.VMEM_SHARED((n_pad, d), jnp.float32),
            pltpu.SemaphoreType.DMA,
            pltpu.SemaphoreType.DMA,
            pltpu.SemaphoreType.DMA,
            pltpu.SemaphoreType.DMA,
        ],
    )
    def body(src_hbm, dst_hbm, val_hbm, x_hbm, sup_hbm,
             zbuf, sidx_all, didx_all, rows0, rows1, vbuf0, vbuf1, acc,
             gsem0, gsem1, vsem0, vsem1):
        cid = lax.axis_index("c")
        sid = lax.axis_index("s")
        row0 = sid * rows_per_tile
        rows = (rows0, rows1)
        vbufs = (vbuf0, vbuf1)
        gsems = (gsem0, gsem1)
        vsems = (vsem0, vsem1)

        zero_v = jnp.zeros((_L,), jnp.float32)

        def zfill(i, _):
            e = i // nsl
            j = i % nsl
            zbuf[e, pl.ds(j * _L, _L)] = zero_v
            return 0

        lax.fori_loop(0, zrows * nsl, zfill, 0)

        for rr in range(r_per_core):
            r = cid * r_per_core + rr
            vbase_rel = (r * _NS + sid) * nbatch * blv

            def zero_acc(z, _):
                pltpu.sync_copy(zbuf, acc.at[pl.ds(row0 + z * zrows, zrows)])
                return 0

            lax.fori_loop(0, rows_per_tile // zrows, zero_acc, 0)

            pltpu.sync_copy(src_hbm.at[r, sid], sidx_all)
            pltpu.sync_copy(dst_hbm.at[r, sid], didx_all)
            plsc.subcore_barrier()

            pltpu.async_copy(x_hbm.at[sidx_all.at[0]], rows0, gsem0)
            pltpu.async_copy(val_hbm.at[pl.ds(vbase_rel, blv)], vbuf0, vsem0)

            def pair_body(g, _):
                for p in range(2):
                    b = 2 * g + p
                    nb = b + 1

                    @pl.when(nb < nbatch)
                    def _():
                        pltpu.async_copy(x_hbm.at[sidx_all.at[nb]],
                                         rows[1 - p], gsems[1 - p])
                        pltpu.async_copy(
                            val_hbm.at[pl.ds(vbase_rel + nb * blv, blv)],
                            vbufs[1 - p], vsems[1 - p])

                    pltpu.make_async_copy(x_hbm.at[sidx_all.at[b]],
                                          rows[p], gsems[p]).wait()
                    pltpu.make_async_copy(
                        val_hbm.at[pl.ds(vbase_rel + b * blv, blv)],
                        vbufs[p], vsems[p]).wait()

                    @plsc.parallel_loop(0, _B, 1, unroll=4)
                    def _(e):
                        vb = vbufs[p][pl.ds(e * _L, _L)]
                        for j in range(nsl):
                            sl = rows[p][e, pl.ds(j * _L, _L)]
                            rows[p][e, pl.ds(j * _L, _L)] = sl * vb

                    pltpu.sync_copy(rows[p], acc.at[didx_all.at[b]], add=True)
                return 0

            lax.fori_loop(0, nbatch // 2, pair_body, 0)
            plsc.subcore_barrier()

            pltpu.sync_copy(acc.at[pl.ds(row0, rows_per_tile)],
                            sup_hbm.at[r, pl.ds(row0, rows_per_tile)])
            plsc.subcore_barrier()

    return body(src, dst, vals_exp, X)


def _tc_matmul(sup, w, n, d, r_total, o):
    bn = 1000
    assert n % bn == 0

    def body(sup_ref, w_ref, out_ref):
        acc = jnp.zeros((bn, o), jnp.float32)
        for r in range(r_total):
            acc = acc + jnp.dot(sup_ref[r], w_ref[r],
                                preferred_element_type=jnp.float32)
        out_ref[...] = acc

    return pl.pallas_call(
        body,
        grid=(n // bn,),
        in_specs=[
            pl.BlockSpec((r_total, bn, d), lambda i: (0, i, 0)),
            pl.BlockSpec((r_total, d, o), lambda i: (0, 0, 0)),
        ],
        out_specs=pl.BlockSpec((bn, o), lambda i: (i, 0)),
        out_shape=jax.ShapeDtypeStruct((n, o), jnp.float32),
    )(sup, w)


def kernel(A_edge_index, A_values, X, w):
    n, d = X.shape
    r_total, _, e = A_edge_index.shape
    o = w.shape[2]

    chunk = _NS * _B * 2
    e_pad = ((e + chunk - 1) // chunk) * chunk
    pad = e_pad - e
    dst = A_edge_index[:, 0, :]
    src = A_edge_index[:, 1, :]
    if pad:
        dst = jnp.pad(dst, ((0, 0), (0, pad)))
        src = jnp.pad(src, ((0, 0), (0, pad)))
        vals = jnp.pad(A_values, ((0, 0), (0, pad)))
    else:
        vals = A_values

    n_pad = ((n + _NS * 16 - 1) // (_NS * 16)) * (_NS * 16)

    vals_exp = jnp.repeat(vals.reshape(-1), _L)

    nbatch = e_pad // (_NS * _B)
    src4 = src.reshape(r_total, _NS, nbatch, _B)
    dst4 = dst.reshape(r_total, _NS, nbatch, _B)

    sup = _sc_supports(src4, dst4, vals_exp, X, n, n_pad, d, r_total, e_pad)
    return _tc_matmul(sup, w, n, d, r_total, o)


# speedup vs baseline: 2.9334x; 1.4898x over previous
import functools

import jax
import jax.numpy as jnp
from jax import lax
from jax.experimental import pallas as pl
from jax.experimental.pallas import tpu as pltpu
from jax.experimental.pallas import tpu_sc as plsc

_NC = 2
_NS = 16
_L = 16

_B = 128


def _sc_supports(src, dst, vals, X, n, n_pad, d, r_total, e_pad):
    r_per_core = r_total // _NC
    e_per_tile = e_pad // _NS
    nbatch = e_per_tile // _B
    assert nbatch % 2 == 0
    rows_per_tile = n_pad // _NS
    nsl = d // _L
    zrows = 16

    mesh = plsc.VectorSubcoreMesh(core_axis_name="c", subcore_axis_name="s")

    @functools.partial(
        pl.kernel,
        out_type=jax.ShapeDtypeStruct((r_total, n_pad, d), jnp.float32),
        mesh=mesh,
        scratch_types=[
            pltpu.VMEM((zrows, d), jnp.float32),
            pltpu.VMEM((nbatch, _B), jnp.int32),
            pltpu.VMEM((nbatch, _B), jnp.int32),
            pltpu.VMEM((_B, d), jnp.float32),
            pltpu.VMEM((_B, d), jnp.float32),
            pltpu.VMEM((_B + _L,), jnp.float32),
            pltpu.VMEM((_B + _L,), jnp.float32),
            pltpu.VMEM_SHARED((n_pad, d), jnp.float32),
            pltpu.SemaphoreType.DMA,
            pltpu.SemaphoreType.DMA,
            pltpu.SemaphoreType.DMA,
            pltpu.SemaphoreType.DMA,
        ],
    )
    def body(src_hbm, dst_hbm, val_hbm, x_hbm, sup_hbm,
             zbuf, sidx_all, didx_all, rows0, rows1, vbuf0, vbuf1, acc,
             gsem0, gsem1, vsem0, vsem1):
        cid = lax.axis_index("c")
        sid = lax.axis_index("s")
        row0 = sid * rows_per_tile
        rows = (rows0, rows1)
        vbufs = (vbuf0, vbuf1)
        gsems = (gsem0, gsem1)
        vsems = (vsem0, vsem1)

        zero_v = jnp.zeros((_L,), jnp.float32)

        def zfill(i, _):
            e = i // nsl
            j = i % nsl
            zbuf[e, pl.ds(j * _L, _L)] = zero_v
            return 0

        lax.fori_loop(0, zrows * nsl, zfill, 0)

        for rr in range(r_per_core):
            r = cid * r_per_core + rr
            vbase_rel = r * e_pad + sid * e_per_tile

            def zero_acc(z, _):
                pltpu.sync_copy(zbuf, acc.at[pl.ds(row0 + z * zrows, zrows)])
                return 0

            lax.fori_loop(0, rows_per_tile // zrows, zero_acc, 0)

            pltpu.sync_copy(src_hbm.at[r, sid], sidx_all)
            pltpu.sync_copy(dst_hbm.at[r, sid], didx_all)
            plsc.subcore_barrier()

            pltpu.async_copy(x_hbm.at[sidx_all.at[0]], rows0, gsem0)
            pltpu.async_copy(val_hbm.at[pl.ds(vbase_rel, _B)],
                             vbuf0.at[pl.ds(0, _B)], vsem0)

            def pair_body(g, _):
                for p in range(2):
                    b = 2 * g + p
                    nb = b + 1

                    @pl.when(nb < nbatch)
                    def _():
                        pltpu.async_copy(x_hbm.at[sidx_all.at[nb]],
                                         rows[1 - p], gsems[1 - p])
                        pltpu.async_copy(
                            val_hbm.at[pl.ds(vbase_rel + nb * _B, _B)],
                            vbufs[1 - p].at[pl.ds(0, _B)], vsems[1 - p])

                    pltpu.make_async_copy(x_hbm.at[sidx_all.at[b]],
                                          rows[p], gsems[p]).wait()
                    pltpu.make_async_copy(
                        val_hbm.at[pl.ds(vbase_rel + b * _B, _B)],
                        vbufs[p].at[pl.ds(0, _B)], vsems[p]).wait()

                    @plsc.parallel_loop(0, _B, 1, unroll=4)
                    def _(e):
                        v = vbufs[p][pl.ds(e, _L)][0]
                        for j in range(nsl):
                            sl = rows[p][e, pl.ds(j * _L, _L)]
                            rows[p][e, pl.ds(j * _L, _L)] = sl * v

                    pltpu.sync_copy(rows[p], acc.at[didx_all.at[b]], add=True)
                return 0

            lax.fori_loop(0, nbatch // 2, pair_body, 0)
            plsc.subcore_barrier()

            pltpu.sync_copy(acc.at[pl.ds(row0, rows_per_tile)],
                            sup_hbm.at[r, pl.ds(row0, rows_per_tile)])
            plsc.subcore_barrier()

    return body(src, dst, vals, X)


def _tc_matmul(sup, w, n, d, r_total, o):
    bn = 1000
    assert n % bn == 0

    def body(sup_ref, w_ref, out_ref):
        acc = jnp.zeros((bn, o), jnp.float32)
        for r in range(r_total):
            acc = acc + jnp.dot(sup_ref[r], w_ref[r],
                                preferred_element_type=jnp.float32)
        out_ref[...] = acc

    return pl.pallas_call(
        body,
        grid=(n // bn,),
        in_specs=[
            pl.BlockSpec((r_total, bn, d), lambda i: (0, i, 0)),
            pl.BlockSpec((r_total, d, o), lambda i: (0, 0, 0)),
        ],
        out_specs=pl.BlockSpec((bn, o), lambda i: (i, 0)),
        out_shape=jax.ShapeDtypeStruct((n, o), jnp.float32),
    )(sup, w)


def kernel(A_edge_index, A_values, X, w):
    n, d = X.shape
    r_total, _, e = A_edge_index.shape
    o = w.shape[2]

    chunk = _NS * _B * 2
    e_pad = ((e + chunk - 1) // chunk) * chunk
    pad = e_pad - e
    dst = A_edge_index[:, 0, :]
    src = A_edge_index[:, 1, :]
    if pad:
        dst = jnp.pad(dst, ((0, 0), (0, pad)))
        src = jnp.pad(src, ((0, 0), (0, pad)))
        vals = jnp.pad(A_values, ((0, 0), (0, pad)))
    else:
        vals = A_values

    n_pad = ((n + _NS * 16 - 1) // (_NS * 16)) * (_NS * 16)

    nbatch = e_pad // (_NS * _B)
    src4 = src.reshape(r_total, _NS, nbatch, _B)
    dst4 = dst.reshape(r_total, _NS, nbatch, _B)

    sup = _sc_supports(src4, dst4, vals.reshape(-1), X, n, n_pad, d,
                       r_total, e_pad)
    return _tc_matmul(sup, w, n, d, r_total, o)
